# baseline (speedup 1.0000x reference)
import functools

import jax
import jax.numpy as jnp
from jax import lax
from jax.experimental import pallas as pl
from jax.experimental.pallas import tpu as pltpu
from jax.experimental.pallas import tpu_sc as plsc

NUM_GRAPHS = 64
NUM_VIRTUAL = 4
N_NODES = 10000
HIDDEN = 128

NC = 2
NS = 16
NW = NC * NS
CHUNK = N_NODES // NW
SUB = 104
NSUB = CHUNK // SUB
TAIL = N_NODES - NW * CHUNK
TAIL_BASE = NW * CHUNK


def _encoder_kernel(x_ref, W_enc_ref, b_enc_ref, nf_ref):
    nf_ref[...] = jnp.maximum(
        jnp.dot(x_ref[...], W_enc_ref[...]) + b_enc_ref[...], 0.0)


def _mlp_kernel(part_ref, W1_ref, b1_ref, W2_ref, b2_ref, Wp1_ref, bp1_ref,
                Wp2_ref, bp2_ref, out_ref):
    seg = (part_ref[0] + part_ref[1]) * (1.0 / NUM_VIRTUAL)
    h = jnp.maximum(jnp.dot(seg, W1_ref[...]) + b1_ref[...], 0.0)
    gf = jnp.dot(h, W2_ref[...]) + b2_ref[...]
    p = jnp.maximum(jnp.dot(gf, Wp1_ref[...]) + bp1_ref[...], 0.0)
    out_ref[...] = jnp.dot(p, Wp2_ref[...]) + bp2_ref[...]


def _sc_segsum_body(nf_hbm, batch_hbm, out_hbm, rows_v, idx0, idx1, idx2,
                    tail_rows, tail_idx, zbuf, shared):
    c = lax.axis_index("c")
    s = lax.axis_index("s")
    wid = s * NC + c
    base = wid * CHUNK
    idxs = (idx0, idx1, idx2)

    @pl.when(s == 0)
    def _zero_shared():
        for i in range(8):
            for j in range(8):
                zbuf[i, pl.ds(j * 16, 16)] = jnp.zeros((16,), jnp.float32)
        for i in range(NUM_GRAPHS // 8):
            pltpu.sync_copy(zbuf, shared.at[pl.ds(i * 8, 8)])

    plsc.subcore_barrier()

    pltpu.sync_copy(nf_hbm.at[pl.ds(base, CHUNK)], rows_v)
    for j in range(NSUB):
        pltpu.sync_copy(batch_hbm.at[pl.ds(base + j * SUB, SUB)], idxs[j])
    for j in range(NSUB):
        pltpu.sync_copy(rows_v.at[pl.ds(j * SUB, SUB)],
                        shared.at[idxs[j]], add=True)

    @pl.when(wid == NW - 1)
    def _tail():
        pltpu.sync_copy(nf_hbm.at[pl.ds(TAIL_BASE, TAIL)], tail_rows)
        pltpu.sync_copy(batch_hbm.at[pl.ds(TAIL_BASE, TAIL)], tail_idx)
        pltpu.sync_copy(tail_rows, shared.at[tail_idx], add=True)

    plsc.subcore_barrier()

    @pl.when(s == 0)
    def _writeout():
        pltpu.sync_copy(shared, out_hbm.at[c])


_sc_segsum = functools.partial(
    pl.kernel,
    out_type=jax.ShapeDtypeStruct((NC, NUM_GRAPHS, HIDDEN), jnp.float32),
    mesh=plsc.VectorSubcoreMesh(core_axis_name="c", subcore_axis_name="s"),
    scratch_types=[
        pltpu.VMEM((CHUNK, HIDDEN), jnp.float32),
        pltpu.VMEM((SUB,), jnp.int32),
        pltpu.VMEM((SUB,), jnp.int32),
        pltpu.VMEM((SUB,), jnp.int32),
        pltpu.VMEM((TAIL, HIDDEN), jnp.float32),
        pltpu.VMEM((TAIL,), jnp.int32),
        pltpu.VMEM((8, HIDDEN), jnp.float32),
        pltpu.VMEM_SHARED((NUM_GRAPHS, HIDDEN), jnp.float32),
    ],
)(_sc_segsum_body)


def kernel(x, edge_index, batch, W_enc, b_enc, W1, b1, W2, b2, Wp1, bp1,
           Wp2, bp2):
    del edge_index
    vmem = pl.BlockSpec(memory_space=pltpu.MemorySpace.VMEM)

    nf = pl.pallas_call(
        _encoder_kernel,
        in_specs=[vmem] * 3,
        out_specs=vmem,
        out_shape=jax.ShapeDtypeStruct((N_NODES, HIDDEN), jnp.float32),
    )(x, W_enc, b_enc.reshape(1, HIDDEN))

    partials = _sc_segsum(nf, batch)

    out = pl.pallas_call(
        _mlp_kernel,
        in_specs=[vmem] * 9,
        out_specs=vmem,
        out_shape=jax.ShapeDtypeStruct((NUM_GRAPHS, 1), jnp.float32),
    )(partials, W1, b1.reshape(1, HIDDEN), W2, b2.reshape(1, HIDDEN),
      Wp1, bp1.reshape(1, HIDDEN), Wp2, bp2.reshape(1, 1))
    return out


# speedup vs baseline: 3.0201x; 3.0201x over previous
import jax
import jax.numpy as jnp
from jax import lax
from jax.experimental import pallas as pl
from jax.experimental.pallas import tpu as pltpu

NUM_GRAPHS = 64
NUM_VIRTUAL = 4
N_NODES = 10000
HIDDEN = 128

CHUNK = 5000


def _fused_kernel(x_hbm, batch_hbm, Wenc_hbm, benc_hbm, W1_hbm, b1_hbm,
                  W2_hbm, b2_hbm, Wp1_hbm, bp1_hbm, Wp2_hbm, bp2_hbm,
                  out_ref,
                  xb0, xb1, batch_v, wenc_v, benc_v, w1_v, b1_v, w2_v, b2_v,
                  wp1_v, bp1_v, wp2_v, bp2_v,
                  sem_x0, sem_x1, sem_enc, sem_batch, sem_mlp):
    x0_cp = pltpu.make_async_copy(x_hbm.at[pl.ds(0, CHUNK), :], xb0, sem_x0)
    x1_cp = pltpu.make_async_copy(x_hbm.at[pl.ds(CHUNK, CHUNK), :], xb1,
                                  sem_x1)
    enc_cps = [pltpu.make_async_copy(Wenc_hbm, wenc_v, sem_enc),
               pltpu.make_async_copy(benc_hbm, benc_v, sem_enc)]
    batch_cp = pltpu.make_async_copy(batch_hbm, batch_v, sem_batch)
    mlp_cps = [pltpu.make_async_copy(s, d, sem_mlp) for s, d in
               ((W1_hbm, w1_v), (b1_hbm, b1_v), (W2_hbm, w2_v),
                (b2_hbm, b2_v), (Wp1_hbm, wp1_v), (bp1_hbm, bp1_v),
                (Wp2_hbm, wp2_v), (bp2_hbm, bp2_v))]

    x0_cp.start()
    for cp in enc_cps:
        cp.start()
    batch_cp.start()
    x1_cp.start()
    for cp in mlp_cps:
        cp.start()

    for cp in enc_cps:
        cp.wait()
    batch_cp.wait()

    onehot_t = (lax.broadcasted_iota(jnp.int32, (NUM_GRAPHS, N_NODES), 0)
                == batch_v[0, :][None, :]).astype(jnp.float32)

    x0_cp.wait()
    nf0 = jnp.maximum(jnp.dot(xb0[...], wenc_v[...]) + benc_v[...], 0.0)
    acc = jnp.dot(onehot_t[:, :CHUNK], nf0)

    x1_cp.wait()
    nf1 = jnp.maximum(jnp.dot(xb1[...], wenc_v[...]) + benc_v[...], 0.0)
    acc = acc + jnp.dot(onehot_t[:, CHUNK:], nf1)

    for cp in mlp_cps:
        cp.wait()
    seg = acc * (1.0 / NUM_VIRTUAL)
    h = jnp.maximum(jnp.dot(seg, w1_v[...]) + b1_v[...], 0.0)
    gf = jnp.dot(h, w2_v[...]) + b2_v[...]
    p = jnp.maximum(jnp.dot(gf, wp1_v[...]) + bp1_v[...], 0.0)
    out_ref[...] = jnp.dot(p, wp2_v[...]) + bp2_v[...]


def kernel(x, edge_index, batch, W_enc, b_enc, W1, b1, W2, b2, Wp1, bp1,
           Wp2, bp2):
    del edge_index
    hbm = pl.BlockSpec(memory_space=pltpu.MemorySpace.HBM)
    out = pl.pallas_call(
        _fused_kernel,
        in_specs=[hbm] * 12,
        out_specs=pl.BlockSpec(memory_space=pltpu.MemorySpace.VMEM),
        out_shape=jax.ShapeDtypeStruct((NUM_GRAPHS, 1), jnp.float32),
        scratch_shapes=[
            pltpu.VMEM((CHUNK, HIDDEN), jnp.float32),
            pltpu.VMEM((CHUNK, HIDDEN), jnp.float32),
            pltpu.VMEM((1, N_NODES), jnp.int32),
            pltpu.VMEM((HIDDEN, HIDDEN), jnp.float32),
            pltpu.VMEM((1, HIDDEN), jnp.float32),
            pltpu.VMEM((HIDDEN, HIDDEN), jnp.float32),
            pltpu.VMEM((1, HIDDEN), jnp.float32),
            pltpu.VMEM((HIDDEN, HIDDEN), jnp.float32),
            pltpu.VMEM((1, HIDDEN), jnp.float32),
            pltpu.VMEM((HIDDEN, HIDDEN), jnp.float32),
            pltpu.VMEM((1, HIDDEN), jnp.float32),
            pltpu.VMEM((HIDDEN, 1), jnp.float32),
            pltpu.VMEM((1, 1), jnp.float32),
            pltpu.SemaphoreType.DMA,
            pltpu.SemaphoreType.DMA,
            pltpu.SemaphoreType.DMA,
            pltpu.SemaphoreType.DMA,
            pltpu.SemaphoreType.DMA,
        ],
    )(x, batch.reshape(1, N_NODES), W_enc, b_enc.reshape(1, HIDDEN),
      W1, b1.reshape(1, HIDDEN), W2, b2.reshape(1, HIDDEN),
      Wp1, bp1.reshape(1, HIDDEN), Wp2, bp2.reshape(1, 1))
    return out
